# baseline (speedup 1.0000x reference)
import functools
import math

import jax
import jax.numpy as jnp
from jax import lax
from jax.experimental import pallas as pl
from jax.experimental.pallas import tpu as pltpu
from jax.experimental.pallas import tpu_sc as plsc

N = 10000
E = 320000
D = 128
H = 4
DK = D // H

NC = 2
NS = 16
NW = NC * NS
EPT = E // NW

CA = 80
NCHA = EPT // CA
FL = 5
CB = 2000
NCHB = EPT // CB

_SCALE = 1.0 / math.sqrt(DK)


def _proj_body(x_ref, qwt_ref, qb_ref, kwt_ref, kb_ref, q_ref, k_ref):
    xb = x_ref[...]
    q = jnp.dot(xb, qwt_ref[...], preferred_element_type=jnp.float32)
    k = jnp.dot(xb, kwt_ref[...], preferred_element_type=jnp.float32)
    q_ref[...] = (q + qb_ref[...]) * _SCALE
    k_ref[...] = k + kb_ref[...]


def _project(x, Qwt, Qb, Kwt, Kb):
    blk = 1000
    grid = N // blk
    return pl.pallas_call(
        _proj_body,
        grid=(grid,),
        in_specs=[
            pl.BlockSpec((blk, D), lambda i: (i, 0)),
            pl.BlockSpec((D, D), lambda i: (0, 0)),
            pl.BlockSpec((D,), lambda i: (0,)),
            pl.BlockSpec((D, D), lambda i: (0, 0)),
            pl.BlockSpec((D,), lambda i: (0,)),
        ],
        out_specs=[
            pl.BlockSpec((blk, D), lambda i: (i, 0)),
            pl.BlockSpec((blk, D), lambda i: (i, 0)),
        ],
        out_shape=[
            jax.ShapeDtypeStruct((N, D), jnp.float32),
            jax.ShapeDtypeStruct((N, D), jnp.float32),
        ],
    )(x, Qwt, Qb, Kwt, Kb)


def _passa_body(q_hbm, k_hbm, e0_hbm, e1_hbm, exp_hbm,
                qsh, e0all, e1all, qr0, kr0, qr1, kr1, qr2, kr2,
                qr3, kr3, expst,
                sq0, sk0, sq1, sk1, sq2, sk2, sq3, sk3):
    sid = lax.axis_index("s")
    wid = sid * NC + lax.axis_index("c")
    base = wid * EPT
    nps = N // NS

    pltpu.sync_copy(q_hbm.at[pl.ds(sid * nps, nps)],
                    qsh.at[pl.ds(sid * nps, nps)])

    pltpu.sync_copy(e0_hbm.at[pl.ds(base, EPT)], e0all)
    pltpu.sync_copy(e1_hbm.at[pl.ds(base, EPT)], e1all)
    plsc.subcore_barrier()

    lanes = lax.iota(jnp.int32, 16)
    bufs = ((qr0, kr0, sq0, sk0), (qr1, kr1, sq1, sk1),
            (qr2, kr2, sq2, sk2), (qr3, kr3, sq3, sk3))

    def issue(c, qr, kr, sq, sk):
        pltpu.async_copy(qsh.at[e0all.at[pl.ds(c * CA, CA)]], qr, sq)
        pltpu.async_copy(k_hbm.at[e1all.at[pl.ds(c * CA, CA)]], kr, sk)

    def wait(c, qr, kr, sq, sk):
        pltpu.make_async_copy(
            qsh.at[e0all.at[pl.ds(c * CA, CA)]], qr, sq).wait()
        pltpu.make_async_copy(
            k_hbm.at[e1all.at[pl.ds(c * CA, CA)]], kr, sk).wait()

    himask = jnp.full((16,), -65536, jnp.int32)

    def compute(c, qr, kr):
        slot = (c % FL) * (CA * H)
        for g in range(CA // 16):
            rows = lanes + g * 16
            for h in range(H):
                def dot_body(j, acc, _h=h, _rows=rows, _qr=qr, _kr=kr):
                    col0 = _h * (DK // 2) + j * 8
                    for dd in range(8):
                        col = jnp.full((16,), col0 + dd, jnp.int32)
                        qw = plsc.load_gather(_qr, [_rows, col])
                        kw = plsc.load_gather(_kr, [_rows, col])
                        qlo = plsc.bitcast(lax.shift_left(qw, 16), jnp.float32)
                        klo = plsc.bitcast(lax.shift_left(kw, 16), jnp.float32)
                        qhi = plsc.bitcast(jnp.bitwise_and(qw, himask),
                                           jnp.float32)
                        khi = plsc.bitcast(jnp.bitwise_and(kw, himask),
                                           jnp.float32)
                        acc = acc + qlo * klo + qhi * khi
                    return acc

                s = lax.fori_loop(0, DK // 16, dot_body,
                                  jnp.zeros((16,), jnp.float32))
                ev = jnp.exp(s)
                plsc.store_scatter(expst, [slot + rows * H + h], ev)

    for p in range(4):
        issue(p, *bufs[p])

    def pair_body(i, _):
        for b in range(4):
            c = i * 4 + b
            qr, kr, sq, sk = bufs[b]
            wait(c, qr, kr, sq, sk)
            compute(c, qr, kr)

            @pl.when((c % FL) == (FL - 1))
            def _flush(_c=c):
                pltpu.sync_copy(
                    expst,
                    exp_hbm.at[pl.ds((base + (_c - (FL - 1)) * CA) * H,
                                     FL * CA * H)])

            @pl.when(c + 4 < NCHA)
            def _prefetch(_c=c, _qr=qr, _kr=kr, _sq=sq, _sk=sk):
                issue(_c + 4, _qr, _kr, _sq, _sk)
        return 0

    lax.fori_loop(0, (NCHA - 1) // 4, pair_body, 0)

    c_last = NCHA - 1
    qr, kr, sq, sk = bufs[c_last % 4]
    wait(c_last, qr, kr, sq, sk)
    compute(c_last, qr, kr)
    pltpu.sync_copy(
        expst,
        exp_hbm.at[pl.ds((base + (NCHA - FL) * CA) * H, FL * CA * H)])


def _passa(q, k, e0, e1):
    mesh = plsc.VectorSubcoreMesh(core_axis_name="c", subcore_axis_name="s")
    kfn = functools.partial(
        pl.kernel,
        mesh=mesh,
        out_type=jax.ShapeDtypeStruct((E * H,), jnp.float32),
        scratch_types=[
            pltpu.VMEM_SHARED((N, D // 2), jnp.int32),
            pltpu.VMEM((EPT,), jnp.int32),
            pltpu.VMEM((EPT,), jnp.int32),
            pltpu.VMEM((CA, D // 2), jnp.int32),
            pltpu.VMEM((CA, D // 2), jnp.int32),
            pltpu.VMEM((CA, D // 2), jnp.int32),
            pltpu.VMEM((CA, D // 2), jnp.int32),
            pltpu.VMEM((CA, D // 2), jnp.int32),
            pltpu.VMEM((CA, D // 2), jnp.int32),
            pltpu.VMEM((CA, D // 2), jnp.int32),
            pltpu.VMEM((CA, D // 2), jnp.int32),
            pltpu.VMEM((FL * CA * H,), jnp.float32),
            pltpu.SemaphoreType.DMA,
            pltpu.SemaphoreType.DMA,
            pltpu.SemaphoreType.DMA,
            pltpu.SemaphoreType.DMA,
            pltpu.SemaphoreType.DMA,
            pltpu.SemaphoreType.DMA,
            pltpu.SemaphoreType.DMA,
            pltpu.SemaphoreType.DMA,
        ],
        compiler_params=pltpu.CompilerParams(needs_layout_passes=False,
                                             use_tc_tiling_on_sc=False),
    )(_passa_body)
    return kfn(q, k, e0, e1)


CD = 2000
NCHD = EPT // CD


def _passden_body(exp_hbm, e0_hbm, pden_hbm, den, e0v, expv):
    wid = lax.axis_index("s") * NC + lax.axis_index("c")
    base = wid * EPT
    lanes = lax.iota(jnp.int32, 16)
    idx4 = jax.lax.shift_right_logical(lanes, 2)
    headv = jnp.bitwise_and(lanes, 3)

    def zero_body(i, _):
        den[pl.ds(i * 16, 16)] = jnp.zeros((16,), jnp.float32)
        return 0

    lax.fori_loop(0, (N * H) // 16, zero_body, 0, unroll=8)

    def chunk_body(ci, _):
        off = base + ci * CD
        pltpu.sync_copy(e0_hbm.at[pl.ds(off, CD)], e0v)
        pltpu.sync_copy(exp_hbm.at[pl.ds(off * H, CD * H)], expv)

        def tbody(t, _2):
            e0rep = plsc.load_gather(e0v, [idx4 + t * 4])
            ev = expv[pl.ds(t * 16, 16)]
            plsc.addupdate_scatter(den, [e0rep * H + headv], ev)
            return 0

        lax.fori_loop(0, (CD * H) // 16, tbody, 0, unroll=8)
        return 0

    lax.fori_loop(0, NCHD, chunk_body, 0)
    pltpu.sync_copy(den, pden_hbm.at[pl.ds(wid * (N * H), N * H)])


def _passden(exp_s, e0):
    mesh = plsc.VectorSubcoreMesh(core_axis_name="c", subcore_axis_name="s")
    kfn = functools.partial(
        pl.kernel,
        mesh=mesh,
        out_type=jax.ShapeDtypeStruct((NW * N * H,), jnp.float32),
        scratch_types=[
            pltpu.VMEM((N * H,), jnp.float32),
            pltpu.VMEM((CD,), jnp.int32),
            pltpu.VMEM((CD * H,), jnp.float32),
        ],
        compiler_params=pltpu.CompilerParams(needs_layout_passes=False,
                                             use_tc_tiling_on_sc=False),
    )(_passden_body)
    return kfn(exp_s, e0)


def _red_body(pden_ref, r_ref):
    r_ref[...] = 1.0 / (jnp.sum(pden_ref[...], axis=0) + 1e-16)


def _reduce(pden):
    return pl.pallas_call(
        _red_body,
        out_shape=jax.ShapeDtypeStruct((N * H,), jnp.float32),
    )(pden)


def _passb_body(exp_hbm, e0_hbm, r_hbm, att_hbm,
                rtab, e0v, expv, attst):
    wid = lax.axis_index("s") * NC + lax.axis_index("c")
    base = wid * EPT
    pltpu.sync_copy(r_hbm, rtab)
    lanes = lax.iota(jnp.int32, 16)
    idx4 = jax.lax.shift_right_logical(lanes, 2)
    headv = jnp.bitwise_and(lanes, 3)

    def chunk_body(ci, _):
        off = base + ci * CB
        pltpu.sync_copy(e0_hbm.at[pl.ds(off, CB)], e0v)
        pltpu.sync_copy(exp_hbm.at[pl.ds(off * H, CB * H)], expv)

        def tbody(t, _2):
            e0rep = plsc.load_gather(e0v, [idx4 + t * 4])
            rv = plsc.load_gather(rtab, [e0rep * H + headv])
            ev = expv[pl.ds(t * 16, 16)]
            attst[pl.ds(t * 16, 16)] = ev * rv
            return 0

        lax.fori_loop(0, (CB * H) // 16, tbody, 0, unroll=8)
        pltpu.sync_copy(attst, att_hbm.at[pl.ds(off * H, CB * H)])
        return 0

    lax.fori_loop(0, NCHB, chunk_body, 0)


def _passb(exp_s, e0, r):
    mesh = plsc.VectorSubcoreMesh(core_axis_name="c", subcore_axis_name="s")
    kfn = functools.partial(
        pl.kernel,
        mesh=mesh,
        out_type=jax.ShapeDtypeStruct((E * H,), jnp.float32),
        scratch_types=[
            pltpu.VMEM((N * H,), jnp.float32),
            pltpu.VMEM((CB,), jnp.int32),
            pltpu.VMEM((CB * H,), jnp.float32),
            pltpu.VMEM((CB * H,), jnp.float32),
        ],
        compiler_params=pltpu.CompilerParams(needs_layout_passes=False),
    )(_passb_body)
    return kfn(exp_s, e0, r)


def _pack_bf16(a):
    a16 = a.astype(jnp.bfloat16).reshape(N, D // 2, 2)
    return jax.lax.bitcast_convert_type(a16, jnp.int32)


def kernel(x, edge, Qw, Qb, Kw, Kb):
    e0 = edge[0].astype(jnp.int32)
    e1 = edge[1].astype(jnp.int32)
    q, k = _project(x, Qw.T, Qb, Kw.T, Kb)
    exp_s = _passa(_pack_bf16(q), _pack_bf16(k), e0, e1)
    pden = _passden(exp_s, e0)
    r = _reduce(pden.reshape(NW, N * H))
    return _passb(exp_s, e0, r).reshape(E, H)


# speedup vs baseline: 1.0087x; 1.0087x over previous
import functools
import math

import jax
import jax.numpy as jnp
from jax import lax
from jax.experimental import pallas as pl
from jax.experimental.pallas import tpu as pltpu
from jax.experimental.pallas import tpu_sc as plsc

N = 10000
E = 320000
D = 128
H = 4
DK = D // H

NC = 2
NS = 16
NW = NC * NS
EPT = E // NW

CA = 400
NCHA = EPT // CA
FL = 1
CB = 2000
NCHB = EPT // CB

_SCALE = 1.0 / math.sqrt(DK)


def _proj_body(x_ref, qwt_ref, qb_ref, kwt_ref, kb_ref, q_ref, k_ref):
    xb = x_ref[...]
    q = jnp.dot(xb, qwt_ref[...], preferred_element_type=jnp.float32)
    k = jnp.dot(xb, kwt_ref[...], preferred_element_type=jnp.float32)
    q_ref[...] = (q + qb_ref[...]) * _SCALE
    k_ref[...] = k + kb_ref[...]


def _project(x, Qwt, Qb, Kwt, Kb):
    blk = 1000
    grid = N // blk
    return pl.pallas_call(
        _proj_body,
        grid=(grid,),
        in_specs=[
            pl.BlockSpec((blk, D), lambda i: (i, 0)),
            pl.BlockSpec((D, D), lambda i: (0, 0)),
            pl.BlockSpec((D,), lambda i: (0,)),
            pl.BlockSpec((D, D), lambda i: (0, 0)),
            pl.BlockSpec((D,), lambda i: (0,)),
        ],
        out_specs=[
            pl.BlockSpec((blk, D), lambda i: (i, 0)),
            pl.BlockSpec((blk, D), lambda i: (i, 0)),
        ],
        out_shape=[
            jax.ShapeDtypeStruct((N, D), jnp.float32),
            jax.ShapeDtypeStruct((N, D), jnp.float32),
        ],
    )(x, Qwt, Qb, Kwt, Kb)


def _passa_body(q_hbm, k_hbm, e0_hbm, e1_hbm, exp_hbm,
                e0all, e1all, qr0, kr0, qr1, kr1, expst,
                sq0, sk0, sq1, sk1):
    wid = lax.axis_index("s") * NC + lax.axis_index("c")
    base = wid * EPT

    pltpu.sync_copy(e0_hbm.at[pl.ds(base, EPT)], e0all)
    pltpu.sync_copy(e1_hbm.at[pl.ds(base, EPT)], e1all)

    lanes = lax.iota(jnp.int32, 16)
    bufs = ((qr0, kr0, sq0, sk0), (qr1, kr1, sq1, sk1))

    def issue(c, qr, kr, sq, sk):
        pltpu.async_copy(q_hbm.at[e0all.at[pl.ds(c * CA, CA)]], qr, sq)
        pltpu.async_copy(k_hbm.at[e1all.at[pl.ds(c * CA, CA)]], kr, sk)

    def wait(c, qr, kr, sq, sk):
        pltpu.make_async_copy(
            q_hbm.at[e0all.at[pl.ds(c * CA, CA)]], qr, sq).wait()
        pltpu.make_async_copy(
            k_hbm.at[e1all.at[pl.ds(c * CA, CA)]], kr, sk).wait()

    himask = jnp.full((16,), -65536, jnp.int32)

    def compute(c, qr, kr):
        def group_body(g, _, _qr=qr, _kr=kr):
            rows = lanes + g * 16
            for h in range(H):
                def dot_body(j, acc, _h=h, _rows=rows, _q=_qr, _k=_kr):
                    col0 = _h * (DK // 2) + j * 8
                    for dd in range(8):
                        col = jnp.full((16,), col0 + dd, jnp.int32)
                        qw = plsc.load_gather(_q, [_rows, col])
                        kw = plsc.load_gather(_k, [_rows, col])
                        qlo = plsc.bitcast(lax.shift_left(qw, 16), jnp.float32)
                        klo = plsc.bitcast(lax.shift_left(kw, 16), jnp.float32)
                        qhi = plsc.bitcast(jnp.bitwise_and(qw, himask),
                                           jnp.float32)
                        khi = plsc.bitcast(jnp.bitwise_and(kw, himask),
                                           jnp.float32)
                        acc = acc + qlo * klo + qhi * khi
                    return acc

                s = lax.fori_loop(0, DK // 16, dot_body,
                                  jnp.zeros((16,), jnp.float32))
                ev = jnp.exp(s)
                plsc.store_scatter(expst, [rows * H + h], ev)
            return 0

        lax.fori_loop(0, CA // 16, group_body, 0)

    for p in range(2):
        issue(p, *bufs[p])

    def pair_body(i, _):
        for b in range(2):
            c = i * 2 + b
            qr, kr, sq, sk = bufs[b]
            wait(c, qr, kr, sq, sk)
            compute(c, qr, kr)

            @pl.when((c % FL) == (FL - 1))
            def _flush(_c=c):
                pltpu.sync_copy(
                    expst,
                    exp_hbm.at[pl.ds((base + (_c - (FL - 1)) * CA) * H,
                                     FL * CA * H)])

            @pl.when(c + 2 < NCHA)
            def _prefetch(_c=c, _qr=qr, _kr=kr, _sq=sq, _sk=sk):
                issue(_c + 2, _qr, _kr, _sq, _sk)
        return 0

    lax.fori_loop(0, (NCHA - 1) // 2, pair_body, 0)

    c_last = NCHA - 1
    qr, kr, sq, sk = bufs[c_last % 2]
    wait(c_last, qr, kr, sq, sk)
    compute(c_last, qr, kr)
    pltpu.sync_copy(
        expst,
        exp_hbm.at[pl.ds((base + (NCHA - FL) * CA) * H, FL * CA * H)])


def _passa(q, k, e0, e1):
    mesh = plsc.VectorSubcoreMesh(core_axis_name="c", subcore_axis_name="s")
    kfn = functools.partial(
        pl.kernel,
        mesh=mesh,
        out_type=jax.ShapeDtypeStruct((E * H,), jnp.float32),
        scratch_types=[
            pltpu.VMEM((EPT,), jnp.int32),
            pltpu.VMEM((EPT,), jnp.int32),
            pltpu.VMEM((CA, D // 2), jnp.int32),
            pltpu.VMEM((CA, D // 2), jnp.int32),
            pltpu.VMEM((CA, D // 2), jnp.int32),
            pltpu.VMEM((CA, D // 2), jnp.int32),
            pltpu.VMEM((FL * CA * H,), jnp.float32),
            pltpu.SemaphoreType.DMA,
            pltpu.SemaphoreType.DMA,
            pltpu.SemaphoreType.DMA,
            pltpu.SemaphoreType.DMA,
        ],
        compiler_params=pltpu.CompilerParams(needs_layout_passes=False,
                                             use_tc_tiling_on_sc=False),
    )(_passa_body)
    return kfn(q, k, e0, e1)


CD = 2000
NCHD = EPT // CD


def _passden_body(exp_hbm, e0_hbm, pden_hbm, den, e0v, expv):
    wid = lax.axis_index("s") * NC + lax.axis_index("c")
    base = wid * EPT
    lanes = lax.iota(jnp.int32, 16)
    idx4 = jax.lax.shift_right_logical(lanes, 2)
    headv = jnp.bitwise_and(lanes, 3)

    def zero_body(i, _):
        den[pl.ds(i * 16, 16)] = jnp.zeros((16,), jnp.float32)
        return 0

    lax.fori_loop(0, (N * H) // 16, zero_body, 0, unroll=8)

    def chunk_body(ci, _):
        off = base + ci * CD
        pltpu.sync_copy(e0_hbm.at[pl.ds(off, CD)], e0v)
        pltpu.sync_copy(exp_hbm.at[pl.ds(off * H, CD * H)], expv)

        def tbody(t, _2):
            e0rep = plsc.load_gather(e0v, [idx4 + t * 4])
            ev = expv[pl.ds(t * 16, 16)]
            plsc.addupdate_scatter(den, [e0rep * H + headv], ev)
            return 0

        lax.fori_loop(0, (CD * H) // 16, tbody, 0, unroll=8)
        return 0

    lax.fori_loop(0, NCHD, chunk_body, 0)
    pltpu.sync_copy(den, pden_hbm.at[pl.ds(wid * (N * H), N * H)])


def _passden(exp_s, e0):
    mesh = plsc.VectorSubcoreMesh(core_axis_name="c", subcore_axis_name="s")
    kfn = functools.partial(
        pl.kernel,
        mesh=mesh,
        out_type=jax.ShapeDtypeStruct((NW * N * H,), jnp.float32),
        scratch_types=[
            pltpu.VMEM((N * H,), jnp.float32),
            pltpu.VMEM((CD,), jnp.int32),
            pltpu.VMEM((CD * H,), jnp.float32),
        ],
        compiler_params=pltpu.CompilerParams(needs_layout_passes=False,
                                             use_tc_tiling_on_sc=False),
    )(_passden_body)
    return kfn(exp_s, e0)


def _red_body(pden_ref, r_ref):
    r_ref[...] = 1.0 / (jnp.sum(pden_ref[...], axis=0) + 1e-16)


def _reduce(pden):
    return pl.pallas_call(
        _red_body,
        out_shape=jax.ShapeDtypeStruct((N * H,), jnp.float32),
    )(pden)


def _passb_body(exp_hbm, e0_hbm, r_hbm, att_hbm,
                rtab, e0v, expv, attst):
    wid = lax.axis_index("s") * NC + lax.axis_index("c")
    base = wid * EPT
    pltpu.sync_copy(r_hbm, rtab)
    lanes = lax.iota(jnp.int32, 16)
    idx4 = jax.lax.shift_right_logical(lanes, 2)
    headv = jnp.bitwise_and(lanes, 3)

    def chunk_body(ci, _):
        off = base + ci * CB
        pltpu.sync_copy(e0_hbm.at[pl.ds(off, CB)], e0v)
        pltpu.sync_copy(exp_hbm.at[pl.ds(off * H, CB * H)], expv)

        def tbody(t, _2):
            e0rep = plsc.load_gather(e0v, [idx4 + t * 4])
            rv = plsc.load_gather(rtab, [e0rep * H + headv])
            ev = expv[pl.ds(t * 16, 16)]
            attst[pl.ds(t * 16, 16)] = ev * rv
            return 0

        lax.fori_loop(0, (CB * H) // 16, tbody, 0, unroll=8)
        pltpu.sync_copy(attst, att_hbm.at[pl.ds(off * H, CB * H)])
        return 0

    lax.fori_loop(0, NCHB, chunk_body, 0)


def _passb(exp_s, e0, r):
    mesh = plsc.VectorSubcoreMesh(core_axis_name="c", subcore_axis_name="s")
    kfn = functools.partial(
        pl.kernel,
        mesh=mesh,
        out_type=jax.ShapeDtypeStruct((E * H,), jnp.float32),
        scratch_types=[
            pltpu.VMEM((N * H,), jnp.float32),
            pltpu.VMEM((CB,), jnp.int32),
            pltpu.VMEM((CB * H,), jnp.float32),
            pltpu.VMEM((CB * H,), jnp.float32),
        ],
        compiler_params=pltpu.CompilerParams(needs_layout_passes=False),
    )(_passb_body)
    return kfn(exp_s, e0, r)


def _pack_bf16(a):
    a16 = a.astype(jnp.bfloat16).reshape(N, D // 2, 2)
    return jax.lax.bitcast_convert_type(a16, jnp.int32)


def kernel(x, edge, Qw, Qb, Kw, Kb):
    e0 = edge[0].astype(jnp.int32)
    e1 = edge[1].astype(jnp.int32)
    q, k = _project(x, Qw.T, Qb, Kw.T, Kb)
    exp_s = _passa(_pack_bf16(q), _pack_bf16(k), e0, e1)
    pden = _passden(exp_s, e0)
    r = _reduce(pden.reshape(NW, N * H))
    return _passb(exp_s, e0, r).reshape(E, H)


# speedup vs baseline: 1.0304x; 1.0216x over previous
import functools
import math

import jax
import jax.numpy as jnp
from jax import lax
from jax.experimental import pallas as pl
from jax.experimental.pallas import tpu as pltpu
from jax.experimental.pallas import tpu_sc as plsc

N = 10000
E = 320000
D = 128
H = 4
DK = D // H

NC = 2
NS = 16
NW = NC * NS
EPT = E // NW

CA = 80
NCHA = EPT // CA
FL = 5
CB = 2000
NCHB = EPT // CB

_SCALE = 1.0 / math.sqrt(DK)


def _proj_body(x_ref, qwt_ref, qb_ref, kwt_ref, kb_ref, q_ref, k_ref):
    xb = x_ref[...]
    q = jnp.dot(xb, qwt_ref[...], preferred_element_type=jnp.float32)
    k = jnp.dot(xb, kwt_ref[...], preferred_element_type=jnp.float32)
    q_ref[...] = (q + qb_ref[...]) * _SCALE
    k_ref[...] = k + kb_ref[...]


def _project(x, Qwt, Qb, Kwt, Kb):
    blk = 1000
    grid = N // blk
    return pl.pallas_call(
        _proj_body,
        grid=(grid,),
        in_specs=[
            pl.BlockSpec((blk, D), lambda i: (i, 0)),
            pl.BlockSpec((D, D), lambda i: (0, 0)),
            pl.BlockSpec((D,), lambda i: (0,)),
            pl.BlockSpec((D, D), lambda i: (0, 0)),
            pl.BlockSpec((D,), lambda i: (0,)),
        ],
        out_specs=[
            pl.BlockSpec((blk, D), lambda i: (i, 0)),
            pl.BlockSpec((blk, D), lambda i: (i, 0)),
        ],
        out_shape=[
            jax.ShapeDtypeStruct((N, D), jnp.float32),
            jax.ShapeDtypeStruct((N, D), jnp.float32),
        ],
    )(x, Qwt, Qb, Kwt, Kb)


def _passa_body(q_hbm, k_hbm, e0_hbm, e1_hbm, exp_hbm, pden_hbm,
                e0all, e1all, qr0, kr0, qr1, kr1, expst, den,
                sq0, sk0, sq1, sk1):
    wid = lax.axis_index("s") * NC + lax.axis_index("c")
    base = wid * EPT

    def zero_body(i, _):
        den[pl.ds(i * 16, 16)] = jnp.zeros((16,), jnp.float32)
        return 0

    lax.fori_loop(0, (N * H) // 16, zero_body, 0, unroll=8)

    pltpu.sync_copy(e0_hbm.at[pl.ds(base, EPT)], e0all)
    pltpu.sync_copy(e1_hbm.at[pl.ds(base, EPT)], e1all)

    lanes = lax.iota(jnp.int32, 16)
    bufs = ((qr0, kr0, sq0, sk0), (qr1, kr1, sq1, sk1))

    def issue(c, qr, kr, sq, sk):
        pltpu.async_copy(q_hbm.at[e0all.at[pl.ds(c * CA, CA)]], qr, sq)
        pltpu.async_copy(k_hbm.at[e1all.at[pl.ds(c * CA, CA)]], kr, sk)

    def wait(c, qr, kr, sq, sk):
        pltpu.make_async_copy(
            q_hbm.at[e0all.at[pl.ds(c * CA, CA)]], qr, sq).wait()
        pltpu.make_async_copy(
            k_hbm.at[e1all.at[pl.ds(c * CA, CA)]], kr, sk).wait()

    himask = jnp.full((16,), -65536, jnp.int32)

    def compute(c, qr, kr):
        slot = (c % FL) * (CA * H)
        for g in range(CA // 16):
            rows = lanes + g * 16
            e0g = e0all[pl.ds(c * CA + g * 16, 16)]
            for h in range(H):
                def dot_body(j, acc, _h=h, _rows=rows, _qr=qr, _kr=kr):
                    col0 = _h * (DK // 2) + j * 8
                    for dd in range(8):
                        col = jnp.full((16,), col0 + dd, jnp.int32)
                        qw = plsc.load_gather(_qr, [_rows, col])
                        kw = plsc.load_gather(_kr, [_rows, col])
                        qlo = plsc.bitcast(lax.shift_left(qw, 16), jnp.float32)
                        klo = plsc.bitcast(lax.shift_left(kw, 16), jnp.float32)
                        qhi = plsc.bitcast(jnp.bitwise_and(qw, himask),
                                           jnp.float32)
                        khi = plsc.bitcast(jnp.bitwise_and(kw, himask),
                                           jnp.float32)
                        acc = acc + qlo * klo + qhi * khi
                    return acc

                s = lax.fori_loop(0, DK // 16, dot_body,
                                  jnp.zeros((16,), jnp.float32))
                ev = jnp.exp(s)
                plsc.store_scatter(expst, [slot + rows * H + h], ev)
                plsc.addupdate_scatter(den, [e0g * H + h], ev)

    issue(0, *bufs[0])
    issue(1, *bufs[1])

    def pair_body(i, _):
        for b in range(2):
            c = i * 2 + b
            qr, kr, sq, sk = bufs[b]
            wait(c, qr, kr, sq, sk)
            compute(c, qr, kr)

            @pl.when((c % FL) == (FL - 1))
            def _flush(_c=c):
                pltpu.sync_copy(
                    expst,
                    exp_hbm.at[pl.ds((base + (_c - (FL - 1)) * CA) * H,
                                     FL * CA * H)])

            @pl.when(c + 2 < NCHA)
            def _prefetch(_c=c, _qr=qr, _kr=kr, _sq=sq, _sk=sk):
                issue(_c + 2, _qr, _kr, _sq, _sk)
        return 0

    lax.fori_loop(0, (NCHA - 1) // 2, pair_body, 0)

    c_last = NCHA - 1
    qr, kr, sq, sk = bufs[c_last % 2]
    wait(c_last, qr, kr, sq, sk)
    compute(c_last, qr, kr)
    pltpu.sync_copy(
        expst,
        exp_hbm.at[pl.ds((base + (NCHA - FL) * CA) * H, FL * CA * H)])
    pltpu.sync_copy(den, pden_hbm.at[pl.ds(wid * (N * H), N * H)])


def _passa(q, k, e0, e1):
    mesh = plsc.VectorSubcoreMesh(core_axis_name="c", subcore_axis_name="s")
    kfn = functools.partial(
        pl.kernel,
        mesh=mesh,
        out_type=[
            jax.ShapeDtypeStruct((E * H,), jnp.float32),
            jax.ShapeDtypeStruct((NW * N * H,), jnp.float32),
        ],
        scratch_types=[
            pltpu.VMEM((EPT,), jnp.int32),
            pltpu.VMEM((EPT,), jnp.int32),
            pltpu.VMEM((CA, D // 2), jnp.int32),
            pltpu.VMEM((CA, D // 2), jnp.int32),
            pltpu.VMEM((CA, D // 2), jnp.int32),
            pltpu.VMEM((CA, D // 2), jnp.int32),
            pltpu.VMEM((FL * CA * H,), jnp.float32),
            pltpu.VMEM((N * H,), jnp.float32),
            pltpu.SemaphoreType.DMA,
            pltpu.SemaphoreType.DMA,
            pltpu.SemaphoreType.DMA,
            pltpu.SemaphoreType.DMA,
        ],
        compiler_params=pltpu.CompilerParams(needs_layout_passes=False,
                                             use_tc_tiling_on_sc=False),
    )(_passa_body)
    return kfn(q, k, e0, e1)


def _red_body(pden_ref, r_ref):
    r_ref[...] = 1.0 / (jnp.sum(pden_ref[...], axis=0) + 1e-16)


def _reduce(pden):
    return pl.pallas_call(
        _red_body,
        out_shape=jax.ShapeDtypeStruct((N * H,), jnp.float32),
    )(pden)


def _passb_body(exp_hbm, e0_hbm, r_hbm, att_hbm,
                rtab, e0v, expv, attst):
    wid = lax.axis_index("s") * NC + lax.axis_index("c")
    base = wid * EPT
    pltpu.sync_copy(r_hbm, rtab)
    lanes = lax.iota(jnp.int32, 16)
    idx4 = jax.lax.shift_right_logical(lanes, 2)
    headv = jnp.bitwise_and(lanes, 3)

    def chunk_body(ci, _):
        off = base + ci * CB
        pltpu.sync_copy(e0_hbm.at[pl.ds(off, CB)], e0v)
        pltpu.sync_copy(exp_hbm.at[pl.ds(off * H, CB * H)], expv)

        def tbody(t, _2):
            e0rep = plsc.load_gather(e0v, [idx4 + t * 4])
            rv = plsc.load_gather(rtab, [e0rep * H + headv])
            ev = expv[pl.ds(t * 16, 16)]
            attst[pl.ds(t * 16, 16)] = ev * rv
            return 0

        lax.fori_loop(0, (CB * H) // 16, tbody, 0, unroll=8)
        pltpu.sync_copy(attst, att_hbm.at[pl.ds(off * H, CB * H)])
        return 0

    lax.fori_loop(0, NCHB, chunk_body, 0)


def _passb(exp_s, e0, r):
    mesh = plsc.VectorSubcoreMesh(core_axis_name="c", subcore_axis_name="s")
    kfn = functools.partial(
        pl.kernel,
        mesh=mesh,
        out_type=jax.ShapeDtypeStruct((E * H,), jnp.float32),
        scratch_types=[
            pltpu.VMEM((N * H,), jnp.float32),
            pltpu.VMEM((CB,), jnp.int32),
            pltpu.VMEM((CB * H,), jnp.float32),
            pltpu.VMEM((CB * H,), jnp.float32),
        ],
        compiler_params=pltpu.CompilerParams(needs_layout_passes=False),
    )(_passb_body)
    return kfn(exp_s, e0, r)


def _pack_bf16(a):
    a16 = a.astype(jnp.bfloat16).reshape(N, D // 2, 2)
    return jax.lax.bitcast_convert_type(a16, jnp.int32)


def kernel(x, edge, Qw, Qb, Kw, Kb):
    e0 = edge[0].astype(jnp.int32)
    e1 = edge[1].astype(jnp.int32)
    q, k = _project(x, Qw.T, Qb, Kw.T, Kb)
    exp_s, pden = _passa(_pack_bf16(q), _pack_bf16(k), e0, e1)
    r = _reduce(pden.reshape(NW, N * H))
    return _passb(exp_s, e0, r).reshape(E, H)
